# baseline (speedup 1.0000x reference)
import functools

import jax
import jax.numpy as jnp
from jax import lax
from jax.experimental import pallas as pl
from jax.experimental.pallas import tpu as pltpu
from jax.experimental.pallas import tpu_sc as plsc

_EMBED = 32
_SEQ = 200
_BATCH = 4096
_PL = _SEQ // 4
_M = _BATCH * 4

_NC = 2
_NS = 16
_NW = _NC * _NS
_BPW = _BATCH // _NW
_SB = 64
_SP = 5
_SM = _SB * 4
_STOK = _SP * _SM
_NBC = _BPW // _SB
_NPC = _PL // _SP
_SLABS = _NBC * _NPC
_PWTOK = _SLABS * _STOK
_SI = 128
_NST = _SM // _SI


@functools.cache
def _make_sc_gather():
    mesh = plsc.VectorSubcoreMesh(core_axis_name="c", subcore_axis_name="s",
                                  num_cores=_NC, num_subcores=_NS)
    return pl.kernel(
        _sc_gather_body,
        out_type=jax.ShapeDtypeStruct((_PL, _M, _EMBED), jnp.float32),
        mesh=mesh,
        scratch_types=[
            pltpu.VMEM((_PWTOK,), jnp.int32),
            pltpu.VMEM((2, _SP, _SM, _EMBED), jnp.float32),
            pltpu.SemaphoreType.DMA,
            pltpu.SemaphoreType.DMA,
        ],
        compiler_params=pltpu.CompilerParams(use_tc_tiling_on_sc=False),
    )


def _sc_gather_body(idx_hbm, table_hbm, out_raw, idx_v, rows_v, gsem, wsem):
    wid = lax.axis_index("s") * _NC + lax.axis_index("c")
    m0 = wid * _BPW * 4
    pltpu.sync_copy(idx_hbm.at[wid], idx_v)

    def slab(si, buf):
        bc = si // _NPC
        pc = si % _NPC

        @pl.when(si >= 2)
        def _():
            pltpu.make_async_copy(
                rows_v.at[buf], out_raw.at[pl.ds(0, _SP), pl.ds(0, _SM), :],
                wsem).wait()

        copies = []
        for jr in range(_SP):
            for h in range(_NST):
                copies.append(pltpu.async_copy(
                    table_hbm.at[idx_v.at[pl.ds(
                        si * _STOK + (jr * _NST + h) * _SI, _SI)]],
                    rows_v.at[buf, jr, pl.ds(h * _SI, _SI), :],
                    gsem))
        for c in copies:
            c.wait()
        pltpu.async_copy(
            rows_v.at[buf],
            out_raw.at[pl.ds(pc * _SP, _SP),
                       pl.ds(m0 + bc * _SM, _SM), :], wsem)

    def body(it, carry):
        slab(it * 2, 0)
        slab(it * 2 + 1, 1)
        return carry

    lax.fori_loop(0, _SLABS // 2, body, 0)
    for buf in (0, 1):
        pltpu.make_async_copy(
            rows_v.at[buf], out_raw.at[pl.ds(0, _SP), pl.ds(0, _SM), :],
            wsem).wait()


_BM = 256


def _mlp_body(x_ref, w1_ref, b1_ref, w2_ref, b2_ref, o_ref, h_ref):
    j = pl.program_id(1)
    p = jnp.dot(x_ref[0], w1_ref[0], preferred_element_type=jnp.float32)

    @pl.when(j == 0)
    def _():
        h_ref[...] = p

    @pl.when(j > 0)
    def _():
        h_ref[...] += p

    @pl.when(j == _PL - 1)
    def _():
        h = jnp.maximum(h_ref[...] + b1_ref[...], 0.0)
        o = jnp.dot(h, w2_ref[...], preferred_element_type=jnp.float32)
        o_ref[...] = 1.0 / (1.0 + jnp.exp(-(o + b2_ref[...])))


def _tc_mlp(emb, W1, b1, W2, b2):
    w1v = W1.reshape(_PL, 128, _EMBED)
    return pl.pallas_call(
        _mlp_body,
        grid=(_BATCH // _BM, _PL),
        in_specs=[
            pl.BlockSpec((1, _BM, 128), lambda i, j: (j, i, 0)),
            pl.BlockSpec((1, 128, _EMBED), lambda i, j: (j, 0, 0)),
            pl.BlockSpec((1, _EMBED), lambda i, j: (0, 0)),
            pl.BlockSpec((_EMBED, 1), lambda i, j: (0, 0)),
            pl.BlockSpec((1, 1), lambda i, j: (0, 0)),
        ],
        out_specs=pl.BlockSpec((_BM, 1), lambda i, j: (i, 0)),
        out_shape=jax.ShapeDtypeStruct((_BATCH, 1), jnp.float32),
        scratch_shapes=[pltpu.VMEM((_BM, _EMBED), jnp.float32)],
    )(emb, w1v, b1.reshape(1, _EMBED), W2, b2.reshape(1, 1))


def kernel(text, table, W1, b1, W2, b2):
    t = text.astype(jnp.int32).reshape(_NW, _NBC, _SB, _NPC, _SP, 4)
    idx2 = t.transpose(0, 1, 3, 4, 2, 5).reshape(_NW, _PWTOK)
    emb = _make_sc_gather()(idx2, table)
    emb3 = emb.reshape(_PL, _BATCH, 128)
    return _tc_mlp(emb3, W1, b1, W2, b2)


# speedup vs baseline: 2.4611x; 2.4611x over previous
import functools

import jax
import jax.numpy as jnp
from jax import lax
from jax.experimental import pallas as pl
from jax.experimental.pallas import tpu as pltpu
from jax.experimental.pallas import tpu_sc as plsc

_EMBED = 32
_SEQ = 200
_BATCH = 4096
_NSPLIT = 2
_BS = _BATCH // _NSPLIT

_NC = 2
_NS = 16
_NW = _NC * _NS
_GI = 128
_KG = 10
_CW = _KG * _GI


@functools.cache
def _make_sc_gather(ntok):
    pw = ntok // _NW
    ng = pw // _GI
    no = ng // _KG
    mesh = plsc.VectorSubcoreMesh(core_axis_name="c", subcore_axis_name="s",
                                  num_cores=_NC, num_subcores=_NS)

    def body(idx_hbm, table_hbm, out_hbm, idx_v, rows_v, gsem, wsem):
        wid = lax.axis_index("s") * _NC + lax.axis_index("c")
        row0 = wid * pw
        pltpu.sync_copy(idx_hbm.at[wid], idx_v)

        def chunk(jj, buf):
            @pl.when(jj >= 2)
            def _():
                pltpu.make_async_copy(
                    rows_v.at[buf], out_hbm.at[pl.ds(row0, _CW)], wsem).wait()

            copies = []
            for g in range(_KG):
                copies.append(pltpu.async_copy(
                    table_hbm.at[idx_v.at[jj * _KG + g]],
                    rows_v.at[buf, pl.ds(g * _GI, _GI), :],
                    gsem))
            for c in copies:
                c.wait()
            pltpu.async_copy(
                rows_v.at[buf], out_hbm.at[pl.ds(row0 + jj * _CW, _CW)], wsem)

        def loop(it, carry):
            chunk(it * 2, 0)
            chunk(it * 2 + 1, 1)
            return carry

        lax.fori_loop(0, no // 2, loop, 0)
        for buf in (0, 1):
            pltpu.make_async_copy(
                rows_v.at[buf], out_hbm.at[pl.ds(row0, _CW)], wsem).wait()

    return pl.kernel(
        body,
        out_type=jax.ShapeDtypeStruct((ntok, _EMBED), jnp.float32),
        mesh=mesh,
        scratch_types=[
            pltpu.VMEM((ng, _GI), jnp.int32),
            pltpu.VMEM((2, _CW, _EMBED), jnp.float32),
            pltpu.SemaphoreType.DMA,
            pltpu.SemaphoreType.DMA,
        ],
        compiler_params=pltpu.CompilerParams(use_tc_tiling_on_sc=False),
    )


_BM = 256


def _mlp_body(x_ref, w1_ref, b1_ref, w2_ref, b2_ref, o_ref):
    h = jnp.dot(x_ref[...], w1_ref[...], preferred_element_type=jnp.float32)
    h = jnp.maximum(h + b1_ref[...], 0.0)
    o = jnp.dot(h, w2_ref[...], preferred_element_type=jnp.float32) + b2_ref[...]
    o_ref[...] = 1.0 / (1.0 + jnp.exp(-o))


def _tc_mlp(flat, W1, b1, W2, b2):
    k = _SEQ * _EMBED
    nb = flat.shape[0]
    return pl.pallas_call(
        _mlp_body,
        grid=(nb // _BM,),
        in_specs=[
            pl.BlockSpec((_BM, k), lambda i: (i, 0)),
            pl.BlockSpec((k, 32), lambda i: (0, 0)),
            pl.BlockSpec((1, 32), lambda i: (0, 0)),
            pl.BlockSpec((32, 1), lambda i: (0, 0)),
            pl.BlockSpec((1, 1), lambda i: (0, 0)),
        ],
        out_specs=pl.BlockSpec((_BM, 1), lambda i: (i, 0)),
        out_shape=jax.ShapeDtypeStruct((nb, 1), jnp.float32),
    )(flat, W1, b1.reshape(1, 32), W2, b2.reshape(1, 1))


def kernel(text, table, W1, b1, W2, b2):
    idx = text.astype(jnp.int32)
    gather = _make_sc_gather(_BS * _SEQ)
    outs = []
    for h in range(_NSPLIT):
        tslice = lax.slice_in_dim(idx, h * _BS, (h + 1) * _BS, axis=0)
        idx3 = tslice.reshape(_NW, (_BS * _SEQ) // (_NW * _GI), _GI)
        emb = gather(idx3, table)
        flat = emb.reshape(_BS, _SEQ * _EMBED)
        outs.append(_tc_mlp(flat, W1, b1, W2, b2))
    return jnp.concatenate(outs, axis=0)


# speedup vs baseline: 2.7617x; 1.1221x over previous
import functools

import jax
import jax.numpy as jnp
from jax import lax
from jax.experimental import pallas as pl
from jax.experimental.pallas import tpu as pltpu
from jax.experimental.pallas import tpu_sc as plsc

_EMBED = 32
_SEQ = 200
_BATCH = 4096
_NSPLIT = 2
_BS = _BATCH // _NSPLIT

_NC = 2
_NS = 16
_NW = _NC * _NS
_GI = 128
_KG = 10
_CW = _KG * _GI


@functools.cache
def _make_sc_gather(ntok):
    pw = ntok // _NW
    ng = pw // _GI
    no = ng // _KG
    mesh = plsc.VectorSubcoreMesh(core_axis_name="c", subcore_axis_name="s",
                                  num_cores=_NC, num_subcores=_NS)

    def body(idx_hbm, table_hbm, out_hbm, idx_v, rows_v, gsem, wsem):
        wid = lax.axis_index("s") * _NC + lax.axis_index("c")
        row0 = wid * pw
        pltpu.sync_copy(idx_hbm.at[wid], idx_v)

        def chunk(jj, buf):
            @pl.when(jj >= 2)
            def _():
                pltpu.make_async_copy(
                    rows_v.at[buf], out_hbm.at[pl.ds(row0, _CW)], wsem).wait()

            copies = []
            for g in range(_KG):
                copies.append(pltpu.async_copy(
                    table_hbm.at[idx_v.at[jj * _KG + g]],
                    rows_v.at[buf, pl.ds(g * _GI, _GI), :],
                    gsem))
            for c in copies:
                c.wait()
            pltpu.async_copy(
                rows_v.at[buf], out_hbm.at[pl.ds(row0 + jj * _CW, _CW)], wsem)

        def loop(it, carry):
            chunk(it * 2, 0)
            chunk(it * 2 + 1, 1)
            return carry

        lax.fori_loop(0, no // 2, loop, 0)
        for buf in (0, 1):
            pltpu.make_async_copy(
                rows_v.at[buf], out_hbm.at[pl.ds(row0, _CW)], wsem).wait()

    return pl.kernel(
        body,
        out_type=jax.ShapeDtypeStruct((ntok, _EMBED), jnp.float32),
        mesh=mesh,
        scratch_types=[
            pltpu.VMEM((ng, _GI), jnp.int32),
            pltpu.VMEM((2, _CW, _EMBED), jnp.float32),
            pltpu.SemaphoreType.DMA,
            pltpu.SemaphoreType.DMA,
        ],
        compiler_params=pltpu.CompilerParams(use_tc_tiling_on_sc=False),
    )


_BM = 256


_RPB = _SEQ * _EMBED // 128


def _mlp_body(x_ref, w1_ref, b1_ref, w2_ref, b2_ref, o_ref):
    x = x_ref[...].reshape(_BM, _SEQ * _EMBED)
    h = jnp.dot(x, w1_ref[...], preferred_element_type=jnp.float32)
    h = jnp.maximum(h + b1_ref[...], 0.0)
    o = jnp.dot(h, w2_ref[...], preferred_element_type=jnp.float32) + b2_ref[...]
    o_ref[...] = 1.0 / (1.0 + jnp.exp(-o))


def _tc_mlp(flat128, W1, b1, W2, b2):
    k = _SEQ * _EMBED
    nb = flat128.shape[0] // _RPB
    return pl.pallas_call(
        _mlp_body,
        grid=(nb // _BM,),
        in_specs=[
            pl.BlockSpec((_BM * _RPB, 128), lambda i: (i, 0)),
            pl.BlockSpec((k, 32), lambda i: (0, 0)),
            pl.BlockSpec((1, 32), lambda i: (0, 0)),
            pl.BlockSpec((32, 1), lambda i: (0, 0)),
            pl.BlockSpec((1, 1), lambda i: (0, 0)),
        ],
        out_specs=pl.BlockSpec((_BM, 1), lambda i: (i, 0)),
        out_shape=jax.ShapeDtypeStruct((nb, 1), jnp.float32),
    )(flat128, W1, b1.reshape(1, 32), W2, b2.reshape(1, 1))


def kernel(text, table, W1, b1, W2, b2):
    idx = text.astype(jnp.int32)
    gather = _make_sc_gather(_BS * _SEQ)
    outs = []
    for h in range(_NSPLIT):
        tslice = lax.slice_in_dim(idx, h * _BS, (h + 1) * _BS, axis=0)
        idx3 = tslice.reshape(_NW, (_BS * _SEQ) // (_NW * _GI), _GI)
        emb = gather(idx3, table)
        flat128 = emb.reshape(_BS * _SEQ * _EMBED // 128, 128)
        outs.append(_tc_mlp(flat128, W1, b1, W2, b2))
    return jnp.concatenate(outs, axis=0)
